# baseline (speedup 1.0000x reference)
import functools
import math

import jax
import jax.numpy as jnp
from jax import lax
from jax.experimental import pallas as pl
from jax.experimental.pallas import tpu as pltpu
from jax.experimental.pallas import tpu_sc as plsc

NF = 128
LENGTH = 10.0
EDGE_BLOCK = 640

N_NODES_K = 10000
N_PAD = 10240
HALF = N_PAD // 2
N_EDGES_K = 160000
C = 10
EC = N_EDGES_K // C
ECROWS = EC // 128
G_NR = ECROWS // 32
G_REM = ECROWS % 32
S_RPT = ECROWS // 16
S_REM = ECROWS % 16
NODES_PER_TILE = N_PAD // 16


def _edge_body(ginv, iedge, gsrc, gdst, dist, edir,
               pw1, pb1, pw2, pb2, ww1, wb1, ww2, wb2,
               payload_out, new_edge_out):
    x = jnp.concatenate([ginv[...], iedge[...]], axis=1)
    h = jnp.dot(x, pw1[...], preferred_element_type=jnp.float32) + pb1[...]
    h = h * jax.nn.sigmoid(h)
    phi = jnp.dot(h, pw2[...], preferred_element_type=jnp.float32) + pb2[...]

    half = NF // 2
    i = jax.lax.broadcasted_iota(jnp.int32, (1, half), 1).astype(jnp.float32)
    freqs = (2.0 * math.pi / LENGTH) * jnp.exp(i * (-math.log(10000.0) / half))
    t = dist[...] * freqs - math.pi
    t2 = t * t
    sinp = t * (9.99984587e-01 + t2 * (-1.66632582e-01 + t2 * (
        8.31238293e-03 + t2 * (-1.93161822e-04 + t2 * 2.17321007e-06))))
    cosp = 9.99999443e-01 + t2 * (-4.99995580e-01 + t2 * (
        4.16610316e-02 + t2 * (-1.38627433e-03 + t2 * (
            2.42531378e-05 + t2 * -2.21936942e-07))))
    pe = jnp.concatenate([-sinp, -cosp], axis=1)
    g = jnp.dot(pe, ww1[...], preferred_element_type=jnp.float32) + wb1[...]
    g = g * jax.nn.sigmoid(g)
    wout = jnp.dot(g, ww2[...], preferred_element_type=jnp.float32) + wb2[...]

    mixed = phi * wout
    gates = mixed[:, 0 * NF:1 * NF]
    cpg = mixed[:, 1 * NF:2 * NF]
    sed = mixed[:, 2 * NF:3 * NF]
    dsv = mixed[:, 3 * NF:4 * NF]
    de = mixed[:, 4 * NF:5 * NF]

    new_edge_out[...] = iedge[...] + de

    ed = edir[...]
    dx = ed[:, 0:1]
    dy = ed[:, 1:2]
    dz = ed[:, 2:3]
    gs = gsrc[...]
    sx = gs[:, 0 * NF:1 * NF]
    sy = gs[:, 1 * NF:2 * NF]
    sz = gs[:, 2 * NF:3 * NF]
    gd = gdst[...]
    tx = gd[:, 0 * NF:1 * NF]
    ty = gd[:, 1 * NF:2 * NF]
    tz = gd[:, 2 * NF:3 * NF]
    cx = dy * tz - dz * ty
    cy = dz * tx - dx * tz
    cz = dx * ty - dy * tx
    payload_out[0, :, :] = sed * dx + gates * sx + cpg * cx
    payload_out[1, :, :] = sed * dy + gates * sy + cpg * cy
    payload_out[2, :, :] = sed * dz + gates * sz + cpg * cz
    payload_out[3, :, :] = dsv


def _edge_compute(ginv, iedge, gsrc, gdst, dist2, edir,
                  pw1, pb1, pw2, pb2, ww1, wb1, ww2, wb2):
    E = iedge.shape[0]
    B = EDGE_BLOCK
    grid = (E // B,)

    def eb(*shape):
        return pl.BlockSpec(shape, lambda i: (i,) + (0,) * (len(shape) - 1))

    def full(a):
        return pl.BlockSpec(a.shape, lambda i: (0,) * a.ndim)

    return pl.pallas_call(
        _edge_body,
        grid=grid,
        in_specs=[
            eb(B, NF), eb(B, NF), eb(B, 3 * NF), eb(B, 3 * NF),
            eb(B, 1), eb(B, 3),
            full(pw1), full(pb1), full(pw2), full(pb2),
            full(ww1), full(wb1), full(ww2), full(wb2),
        ],
        out_specs=[
            pl.BlockSpec((4, B, NF), lambda i: (0, i, 0)),
            eb(B, NF),
        ],
        out_shape=[
            jax.ShapeDtypeStruct((4, E, NF), jnp.float32),
            jax.ShapeDtypeStruct((E, NF), jnp.float32),
        ],
    )(ginv, iedge, gsrc, gdst, dist2, edir,
      pw1, pb1, pw2, pb2, ww1, wb1, ww2, wb2)


def _gather_body(inv_hbm, equiv_hbm, src_hbm, dstr_hbm,
                 ginv_hbm, gsrc_hbm, gdst_hbm,
                 ibuf_src, ibuf_dst, buf128, buf384, gsem):
    c = lax.axis_index("c")
    s = lax.axis_index("s")
    wid = c * 16 + s
    row_base = G_NR * wid + jnp.minimum(wid, G_REM)
    nr = G_NR + jnp.where(wid < G_REM, 1, 0)
    nq = 2 * nr

    pltpu.sync_copy(src_hbm.at[pl.ds(row_base, G_NR)],
                    ibuf_src.at[pl.ds(0, G_NR)])
    pltpu.sync_copy(dstr_hbm.at[pl.ds(row_base, G_NR)],
                    ibuf_dst.at[pl.ds(0, G_NR)])

    @pl.when(wid < G_REM)
    def _():
        pltpu.sync_copy(src_hbm.at[pl.ds(row_base + G_NR, 1)],
                        ibuf_src.at[pl.ds(G_NR, 1)])
        pltpu.sync_copy(dstr_hbm.at[pl.ds(row_base + G_NR, 1)],
                        ibuf_dst.at[pl.ds(G_NR, 1)])

    def phase(table_hbm, ibuf, buf, out_hbm):
        def idx_slc(q):
            return ibuf.at[lax.shift_right_logical(q, 1),
                           pl.ds(64 * lax.rem(q, 2), 64)]

        pltpu.async_copy(table_hbm.at[idx_slc(0)], buf.at[0], gsem)

        def body(q, carry):
            b = lax.rem(q, 2)
            pltpu.make_async_copy(table_hbm.at[idx_slc(q)], buf.at[b],
                                  gsem).wait()

            @pl.when(q + 1 < nq)
            def _():
                pltpu.async_copy(table_hbm.at[idx_slc(q + 1)],
                                 buf.at[1 - b], gsem)

            pltpu.sync_copy(buf.at[b],
                            out_hbm.at[pl.ds(row_base * 128 + q * 64, 64)])
            return carry

        lax.fori_loop(0, nq, body, 0)

    phase(inv_hbm, ibuf_src, buf128, ginv_hbm)
    phase(equiv_hbm, ibuf_src, buf384, gsrc_hbm)
    phase(equiv_hbm, ibuf_dst, buf384, gdst_hbm)


@functools.partial(
    pl.kernel,
    out_type=(
        jax.ShapeDtypeStruct((EC, NF), jnp.float32),
        jax.ShapeDtypeStruct((EC, 3 * NF), jnp.float32),
        jax.ShapeDtypeStruct((EC, 3 * NF), jnp.float32),
    ),
    mesh=plsc.VectorSubcoreMesh(core_axis_name="c", subcore_axis_name="s",
                                num_cores=2, num_subcores=16),
    compiler_params=pltpu.CompilerParams(use_tc_tiling_on_sc=False),
    scratch_types=[
        pltpu.VMEM((G_NR + 1, 128), jnp.int32),
        pltpu.VMEM((G_NR + 1, 128), jnp.int32),
        pltpu.VMEM((2, 64, NF), jnp.float32),
        pltpu.VMEM((2, 64, 3 * NF), jnp.float32),
        pltpu.SemaphoreType.DMA,
    ],
)
def _sc_gather(inv_hbm, equiv_hbm, src_hbm, dstr_hbm,
               ginv_hbm, gsrc_hbm, gdst_hbm,
               ibuf_src, ibuf_dst, buf128, buf384, gsem):
    _gather_body(inv_hbm, equiv_hbm, src_hbm, dstr_hbm,
                 ginv_hbm, gsrc_hbm, gdst_hbm,
                 ibuf_src, ibuf_dst, buf128, buf384, gsem)


def _scatter_body(payload_hbm, dst_hbm, init_hbm, out_hbm,
                  idx_all, pbuf, bbuf, acc, lsem):
    c = lax.axis_index("c")
    s = lax.axis_index("s")
    row_base = S_RPT * s + jnp.minimum(s, S_REM)
    nb = S_RPT + jnp.where(s < S_REM, 1, 0)
    node_base = s * NODES_PER_TILE

    pltpu.sync_copy(dst_hbm.at[pl.ds(row_base, S_RPT)],
                    idx_all.at[pl.ds(0, S_RPT)])

    @pl.when(s < S_REM)
    def _():
        pltpu.sync_copy(dst_hbm.at[pl.ds(row_base + S_RPT, 1)],
                        idx_all.at[pl.ds(S_RPT, 1)])

    nsl = pl.ds(node_base, NODES_PER_TILE)
    for k in range(2):
        j = 2 * c + k
        for hc in range(2):
            csl = pl.ds(64 * hc, 64)
            pltpu.sync_copy(init_hbm.at[j, nsl, csl], bbuf)
            pltpu.sync_copy(bbuf, acc.at[nsl])
            plsc.subcore_barrier()

            pltpu.async_copy(
                payload_hbm.at[j, pl.ds(row_base * 128, 128), csl],
                pbuf.at[0], lsem)

            def body(g, carry):
                buf = lax.rem(g, 2)
                pltpu.make_async_copy(payload_hbm.at[j, pl.ds(0, 128), csl],
                                      pbuf.at[buf], lsem).wait()

                @pl.when(g + 1 < nb)
                def _():
                    pltpu.async_copy(
                        payload_hbm.at[j, pl.ds((row_base + g + 1) * 128, 128),
                                       csl],
                        pbuf.at[1 - buf], lsem)

                pltpu.sync_copy(pbuf.at[buf], acc.at[idx_all.at[g]],
                                add=True)
                return carry

            lax.fori_loop(0, nb, body, 0)
            plsc.subcore_barrier()

            pltpu.sync_copy(acc.at[nsl], bbuf)
            pltpu.sync_copy(bbuf, out_hbm.at[j, nsl, csl])
            plsc.subcore_barrier()


@functools.partial(
    pl.kernel,
    out_type=jax.ShapeDtypeStruct((4, N_PAD, NF), jnp.float32),
    mesh=plsc.VectorSubcoreMesh(core_axis_name="c", subcore_axis_name="s",
                                num_cores=2, num_subcores=16),
    compiler_params=pltpu.CompilerParams(use_tc_tiling_on_sc=False),
    scratch_types=[
        pltpu.VMEM((S_RPT + 1, 128), jnp.int32),
        pltpu.VMEM((2, 128, 64), jnp.float32),
        pltpu.VMEM((NODES_PER_TILE, 64), jnp.float32),
        pltpu.VMEM_SHARED((N_PAD, 64), jnp.float32),
        pltpu.SemaphoreType.DMA,
    ],
)
def _sc_scatter(payload_hbm, dst_hbm, init_hbm, out_hbm,
                idx_all, pbuf, bbuf, acc, lsem):
    _scatter_body(payload_hbm, dst_hbm, init_hbm, out_hbm,
                  idx_all, pbuf, bbuf, acc, lsem)


def kernel(invariant_node_features, equivariant_node_features,
           invariant_edge_features, edge_dist, edge_dir,
           phi_w1, phi_b1, phi_w2, phi_b2,
           w_w1, w_b1, w_w2, w_b2,
           edge_index):
    N = invariant_node_features.shape[0]
    E = invariant_edge_features.shape[0]
    src = edge_index[0]
    dst = edge_index[1]

    equiv_cm = equivariant_node_features.transpose(0, 2, 1).reshape(N, 3 * NF)
    src2d = src.reshape(E // 128, 128)
    dst2d = dst.reshape(E // 128, 128)

    init4 = jnp.concatenate(
        [equiv_cm.reshape(N, 3, NF).transpose(1, 0, 2),
         invariant_node_features[None]], axis=0)
    acc_state = jnp.pad(init4, ((0, 0), (0, N_PAD - N), (0, 0)))

    pb1 = phi_b1.reshape(1, NF)
    pb2 = phi_b2.reshape(1, 5 * NF)
    wb1 = w_b1.reshape(1, NF)
    wb2 = w_b2.reshape(1, 5 * NF)
    dist2 = edge_dist.reshape(E, 1)

    new_edges = []
    for k in range(C):
        esl = slice(k * EC, (k + 1) * EC)
        rsl = slice(k * ECROWS, (k + 1) * ECROWS)
        ginv, gsrc, gdst = _sc_gather(invariant_node_features, equiv_cm,
                                      src2d[rsl], dst2d[rsl])
        payload4, ne = _edge_compute(
            ginv, invariant_edge_features[esl], gsrc, gdst,
            dist2[esl], edge_dir[esl],
            phi_w1, pb1, phi_w2, pb2, w_w1, wb1, w_w2, wb2)
        new_edges.append(ne)
        acc_state = _sc_scatter(payload4, dst2d[rsl], acc_state)

    out4 = acc_state[:, :N, :]
    new_equivariant = out4[:3].transpose(1, 2, 0)
    new_invariant = out4[3]
    new_edge = jnp.concatenate(new_edges, axis=0) if C > 1 else new_edges[0]
    return (new_equivariant, new_invariant, new_edge)


# speedup vs baseline: 1.0592x; 1.0592x over previous
import functools
import math

import jax
import jax.numpy as jnp
from jax import lax
from jax.experimental import pallas as pl
from jax.experimental.pallas import tpu as pltpu
from jax.experimental.pallas import tpu_sc as plsc

NF = 128
LENGTH = 10.0
EDGE_BLOCK = 640

N_NODES_K = 10000
N_PAD = 10240
HALF = N_PAD // 2
N_EDGES_K = 160000
C = 5
EC = N_EDGES_K // C
ECROWS = EC // 128
G_NR = ECROWS // 32
G_REM = ECROWS % 32
S_RPT = ECROWS // 16
S_REM = ECROWS % 16
NODES_PER_TILE = N_PAD // 16


def _edge_body(ginv, iedge, gsrc, gdst, dist, edir,
               pw1, pb1, pw2, pb2, ww1, wb1, ww2, wb2,
               payload_out, new_edge_out):
    x = jnp.concatenate([ginv[...], iedge[...]], axis=1)
    h = jnp.dot(x, pw1[...], preferred_element_type=jnp.float32) + pb1[...]
    h = h * jax.nn.sigmoid(h)
    phi = jnp.dot(h, pw2[...], preferred_element_type=jnp.float32) + pb2[...]

    half = NF // 2
    i = jax.lax.broadcasted_iota(jnp.int32, (1, half), 1).astype(jnp.float32)
    freqs = (2.0 * math.pi / LENGTH) * jnp.exp(i * (-math.log(10000.0) / half))
    t = dist[...] * freqs - math.pi
    t2 = t * t
    sinp = t * (9.99984587e-01 + t2 * (-1.66632582e-01 + t2 * (
        8.31238293e-03 + t2 * (-1.93161822e-04 + t2 * 2.17321007e-06))))
    cosp = 9.99999443e-01 + t2 * (-4.99995580e-01 + t2 * (
        4.16610316e-02 + t2 * (-1.38627433e-03 + t2 * (
            2.42531378e-05 + t2 * -2.21936942e-07))))
    pe = jnp.concatenate([-sinp, -cosp], axis=1)
    g = jnp.dot(pe, ww1[...], preferred_element_type=jnp.float32) + wb1[...]
    g = g * jax.nn.sigmoid(g)
    wout = jnp.dot(g, ww2[...], preferred_element_type=jnp.float32) + wb2[...]

    mixed = phi * wout
    gates = mixed[:, 0 * NF:1 * NF]
    cpg = mixed[:, 1 * NF:2 * NF]
    sed = mixed[:, 2 * NF:3 * NF]
    dsv = mixed[:, 3 * NF:4 * NF]
    de = mixed[:, 4 * NF:5 * NF]

    new_edge_out[...] = iedge[...] + de

    ed = edir[...]
    dx = ed[:, 0:1]
    dy = ed[:, 1:2]
    dz = ed[:, 2:3]
    gs = gsrc[...]
    sx = gs[:, 0 * NF:1 * NF]
    sy = gs[:, 1 * NF:2 * NF]
    sz = gs[:, 2 * NF:3 * NF]
    gd = gdst[...]
    tx = gd[:, 0 * NF:1 * NF]
    ty = gd[:, 1 * NF:2 * NF]
    tz = gd[:, 2 * NF:3 * NF]
    cx = dy * tz - dz * ty
    cy = dz * tx - dx * tz
    cz = dx * ty - dy * tx
    payload_out[0, :, :] = sed * dx + gates * sx + cpg * cx
    payload_out[1, :, :] = sed * dy + gates * sy + cpg * cy
    payload_out[2, :, :] = sed * dz + gates * sz + cpg * cz
    payload_out[3, :, :] = dsv


def _edge_compute(ginv, iedge, gsrc, gdst, dist2, edir,
                  pw1, pb1, pw2, pb2, ww1, wb1, ww2, wb2):
    E = iedge.shape[0]
    B = EDGE_BLOCK
    grid = (E // B,)

    def eb(*shape):
        return pl.BlockSpec(shape, lambda i: (i,) + (0,) * (len(shape) - 1))

    def full(a):
        return pl.BlockSpec(a.shape, lambda i: (0,) * a.ndim)

    return pl.pallas_call(
        _edge_body,
        grid=grid,
        in_specs=[
            eb(B, NF), eb(B, NF), eb(B, 3 * NF), eb(B, 3 * NF),
            eb(B, 1), eb(B, 3),
            full(pw1), full(pb1), full(pw2), full(pb2),
            full(ww1), full(wb1), full(ww2), full(wb2),
        ],
        out_specs=[
            pl.BlockSpec((4, B, NF), lambda i: (0, i, 0)),
            eb(B, NF),
        ],
        out_shape=[
            jax.ShapeDtypeStruct((4, E, NF), jnp.float32),
            jax.ShapeDtypeStruct((E, NF), jnp.float32),
        ],
    )(ginv, iedge, gsrc, gdst, dist2, edir,
      pw1, pb1, pw2, pb2, ww1, wb1, ww2, wb2)


def _gather_body(inv_hbm, equiv_hbm, src_hbm, dstr_hbm,
                 ginv_hbm, gsrc_hbm, gdst_hbm,
                 ibuf_src, ibuf_dst, buf128, buf384, gsem):
    c = lax.axis_index("c")
    s = lax.axis_index("s")
    wid = c * 16 + s
    row_base = G_NR * wid + jnp.minimum(wid, G_REM)
    nr = G_NR + jnp.where(wid < G_REM, 1, 0)
    nq = 2 * nr

    pltpu.sync_copy(src_hbm.at[pl.ds(row_base, G_NR)],
                    ibuf_src.at[pl.ds(0, G_NR)])
    pltpu.sync_copy(dstr_hbm.at[pl.ds(row_base, G_NR)],
                    ibuf_dst.at[pl.ds(0, G_NR)])

    @pl.when(wid < G_REM)
    def _():
        pltpu.sync_copy(src_hbm.at[pl.ds(row_base + G_NR, 1)],
                        ibuf_src.at[pl.ds(G_NR, 1)])
        pltpu.sync_copy(dstr_hbm.at[pl.ds(row_base + G_NR, 1)],
                        ibuf_dst.at[pl.ds(G_NR, 1)])

    def phase(table_hbm, ibuf, buf, out_hbm):
        def idx_slc(q):
            return ibuf.at[lax.shift_right_logical(q, 1),
                           pl.ds(64 * lax.rem(q, 2), 64)]

        pltpu.async_copy(table_hbm.at[idx_slc(0)], buf.at[0], gsem)

        def body(q, carry):
            b = lax.rem(q, 2)
            pltpu.make_async_copy(table_hbm.at[idx_slc(q)], buf.at[b],
                                  gsem).wait()

            @pl.when(q + 1 < nq)
            def _():
                pltpu.async_copy(table_hbm.at[idx_slc(q + 1)],
                                 buf.at[1 - b], gsem)

            pltpu.sync_copy(buf.at[b],
                            out_hbm.at[pl.ds(row_base * 128 + q * 64, 64)])
            return carry

        lax.fori_loop(0, nq, body, 0)

    phase(inv_hbm, ibuf_src, buf128, ginv_hbm)
    phase(equiv_hbm, ibuf_src, buf384, gsrc_hbm)
    phase(equiv_hbm, ibuf_dst, buf384, gdst_hbm)


@functools.partial(
    pl.kernel,
    out_type=(
        jax.ShapeDtypeStruct((EC, NF), jnp.float32),
        jax.ShapeDtypeStruct((EC, 3 * NF), jnp.float32),
        jax.ShapeDtypeStruct((EC, 3 * NF), jnp.float32),
    ),
    mesh=plsc.VectorSubcoreMesh(core_axis_name="c", subcore_axis_name="s",
                                num_cores=2, num_subcores=16),
    compiler_params=pltpu.CompilerParams(use_tc_tiling_on_sc=False),
    scratch_types=[
        pltpu.VMEM((G_NR + 1, 128), jnp.int32),
        pltpu.VMEM((G_NR + 1, 128), jnp.int32),
        pltpu.VMEM((2, 64, NF), jnp.float32),
        pltpu.VMEM((2, 64, 3 * NF), jnp.float32),
        pltpu.SemaphoreType.DMA,
    ],
)
def _sc_gather(inv_hbm, equiv_hbm, src_hbm, dstr_hbm,
               ginv_hbm, gsrc_hbm, gdst_hbm,
               ibuf_src, ibuf_dst, buf128, buf384, gsem):
    _gather_body(inv_hbm, equiv_hbm, src_hbm, dstr_hbm,
                 ginv_hbm, gsrc_hbm, gdst_hbm,
                 ibuf_src, ibuf_dst, buf128, buf384, gsem)


def _scatter_body(payload_hbm, dst_hbm, init_hbm, out_hbm,
                  idx_all, pbuf, bbuf, acc, lsem):
    c = lax.axis_index("c")
    s = lax.axis_index("s")
    row_base = S_RPT * s + jnp.minimum(s, S_REM)
    nb = S_RPT + jnp.where(s < S_REM, 1, 0)
    node_base = s * NODES_PER_TILE

    pltpu.sync_copy(dst_hbm.at[pl.ds(row_base, S_RPT)],
                    idx_all.at[pl.ds(0, S_RPT)])

    @pl.when(s < S_REM)
    def _():
        pltpu.sync_copy(dst_hbm.at[pl.ds(row_base + S_RPT, 1)],
                        idx_all.at[pl.ds(S_RPT, 1)])

    nsl = pl.ds(node_base, NODES_PER_TILE)
    for k in range(2):
        j = 2 * c + k
        for hc in range(2):
            csl = pl.ds(64 * hc, 64)
            pltpu.sync_copy(init_hbm.at[j, nsl, csl], bbuf)
            pltpu.sync_copy(bbuf, acc.at[nsl])
            plsc.subcore_barrier()

            pltpu.async_copy(
                payload_hbm.at[j, pl.ds(row_base * 128, 128), csl],
                pbuf.at[0], lsem)

            def body(g, carry):
                buf = lax.rem(g, 2)
                pltpu.make_async_copy(payload_hbm.at[j, pl.ds(0, 128), csl],
                                      pbuf.at[buf], lsem).wait()

                @pl.when(g + 1 < nb)
                def _():
                    pltpu.async_copy(
                        payload_hbm.at[j, pl.ds((row_base + g + 1) * 128, 128),
                                       csl],
                        pbuf.at[1 - buf], lsem)

                pltpu.sync_copy(pbuf.at[buf], acc.at[idx_all.at[g]],
                                add=True)
                return carry

            lax.fori_loop(0, nb, body, 0)
            plsc.subcore_barrier()

            pltpu.sync_copy(acc.at[nsl], bbuf)
            pltpu.sync_copy(bbuf, out_hbm.at[j, nsl, csl])
            plsc.subcore_barrier()


@functools.partial(
    pl.kernel,
    out_type=jax.ShapeDtypeStruct((4, N_PAD, NF), jnp.float32),
    mesh=plsc.VectorSubcoreMesh(core_axis_name="c", subcore_axis_name="s",
                                num_cores=2, num_subcores=16),
    compiler_params=pltpu.CompilerParams(use_tc_tiling_on_sc=False),
    scratch_types=[
        pltpu.VMEM((S_RPT + 1, 128), jnp.int32),
        pltpu.VMEM((2, 128, 64), jnp.float32),
        pltpu.VMEM((NODES_PER_TILE, 64), jnp.float32),
        pltpu.VMEM_SHARED((N_PAD, 64), jnp.float32),
        pltpu.SemaphoreType.DMA,
    ],
)
def _sc_scatter(payload_hbm, dst_hbm, init_hbm, out_hbm,
                idx_all, pbuf, bbuf, acc, lsem):
    _scatter_body(payload_hbm, dst_hbm, init_hbm, out_hbm,
                  idx_all, pbuf, bbuf, acc, lsem)


def kernel(invariant_node_features, equivariant_node_features,
           invariant_edge_features, edge_dist, edge_dir,
           phi_w1, phi_b1, phi_w2, phi_b2,
           w_w1, w_b1, w_w2, w_b2,
           edge_index):
    N = invariant_node_features.shape[0]
    E = invariant_edge_features.shape[0]
    src = edge_index[0]
    dst = edge_index[1]

    equiv_cm = equivariant_node_features.transpose(0, 2, 1).reshape(N, 3 * NF)
    src2d = src.reshape(E // 128, 128)
    dst2d = dst.reshape(E // 128, 128)

    init4 = jnp.concatenate(
        [equiv_cm.reshape(N, 3, NF).transpose(1, 0, 2),
         invariant_node_features[None]], axis=0)
    acc_state = jnp.pad(init4, ((0, 0), (0, N_PAD - N), (0, 0)))

    pb1 = phi_b1.reshape(1, NF)
    pb2 = phi_b2.reshape(1, 5 * NF)
    wb1 = w_b1.reshape(1, NF)
    wb2 = w_b2.reshape(1, 5 * NF)
    dist2 = edge_dist.reshape(E, 1)

    new_edges = []
    for k in range(C):
        esl = slice(k * EC, (k + 1) * EC)
        rsl = slice(k * ECROWS, (k + 1) * ECROWS)
        ginv, gsrc, gdst = _sc_gather(invariant_node_features, equiv_cm,
                                      src2d[rsl], dst2d[rsl])
        payload4, ne = _edge_compute(
            ginv, invariant_edge_features[esl], gsrc, gdst,
            dist2[esl], edge_dir[esl],
            phi_w1, pb1, phi_w2, pb2, w_w1, wb1, w_w2, wb2)
        new_edges.append(ne)
        acc_state = _sc_scatter(payload4, dst2d[rsl], acc_state)

    out4 = acc_state[:, :N, :]
    new_equivariant = out4[:3].transpose(1, 2, 0)
    new_invariant = out4[3]
    new_edge = jnp.concatenate(new_edges, axis=0) if C > 1 else new_edges[0]
    return (new_equivariant, new_invariant, new_edge)
